# Initial kernel scaffold; baseline (speedup 1.0000x reference)
import jax
import jax.numpy as jnp
from jax.experimental import pallas as pl


def kernel(phonetic_token, semantic_embed, emb_table, W_ph, b_ph, W_se, b_se, W_c, b_c, W_o, b_o):
    raise NotImplementedError("write your pallas kernel here")


# speedup vs baseline: 2.3993x; 2.3993x over previous
import functools

import jax
import jax.numpy as jnp
from jax import lax
from jax.experimental import pallas as pl
from jax.experimental.pallas import tpu as pltpu
from jax.experimental.pallas import tpu_sc as plsc

_B = 16384
_L = 200
_EMB = 64
_NC, _NS = 2, 16
_NW = _NC * _NS
_ROWS_PER_W = _B // _NW
_CHUNK = 100
_CPR = _L // _CHUNK
_G = 16
_CPG = _G * _CPR
_GROUPS = _ROWS_PER_W // _G
_INV_L = 1.0 / _L


def _pool_body(idx_hbm, table_hbm, out_hbm, idx_v, buf0, buf1, out_v,
               sem0, sem1):
    wid = lax.axis_index("s") * _NC + lax.axis_index("c")
    row0 = wid * _ROWS_PER_W

    def accum(buf):
        def body(j, accs):
            a0, a1, a2, a3 = accs
            a0 = a0 + buf[j, 0:16]
            a1 = a1 + buf[j, 16:32]
            a2 = a2 + buf[j, 32:48]
            a3 = a3 + buf[j, 48:64]
            return (a0, a1, a2, a3)
        z = jnp.zeros((16,), jnp.float32)
        return lax.fori_loop(0, _CHUNK, body, (z, z, z, z), unroll=4)

    def group(g, _):
        chunk0 = (row0 + g * _G) * _CPR
        pltpu.sync_copy(idx_hbm.at[pl.ds(chunk0, _CPG), :], idx_v)
        pltpu.async_copy(table_hbm.at[idx_v.at[0]], buf0, sem0)
        pltpu.async_copy(table_hbm.at[idx_v.at[1]], buf1, sem1)

        def row(r, _):
            pltpu.make_async_copy(table_hbm.at[idx_v.at[0]], buf0, sem0).wait()
            a0, a1, a2, a3 = accum(buf0)

            @pl.when(2 * r + 2 < _CPG)
            def _():
                pltpu.async_copy(table_hbm.at[idx_v.at[2 * r + 2]], buf0, sem0)

            pltpu.make_async_copy(table_hbm.at[idx_v.at[1]], buf1, sem1).wait()
            b0, b1, b2, b3 = accum(buf1)

            @pl.when(2 * r + 3 < _CPG)
            def _():
                pltpu.async_copy(table_hbm.at[idx_v.at[2 * r + 3]], buf1, sem1)

            out_v[r, 0:16] = a0 + b0
            out_v[r, 16:32] = a1 + b1
            out_v[r, 32:48] = a2 + b2
            out_v[r, 48:64] = a3 + b3
            return 0

        lax.fori_loop(0, _G, row, 0)
        pltpu.sync_copy(out_v, out_hbm.at[pl.ds(row0 + g * _G, _G), :])
        return 0

    lax.fori_loop(0, _GROUPS, group, 0)


@jax.jit
def _pool(idx, table):
    mesh = plsc.VectorSubcoreMesh(core_axis_name="c", subcore_axis_name="s")
    return pl.kernel(
        _pool_body,
        out_type=jax.ShapeDtypeStruct((_B, _EMB), jnp.float32),
        mesh=mesh,
        compiler_params=pltpu.CompilerParams(use_tc_tiling_on_sc=False),
        scratch_types=[
            pltpu.VMEM((_CPG, _CHUNK), jnp.int32),
            pltpu.VMEM((_CHUNK, _EMB), jnp.float32),
            pltpu.VMEM((_CHUNK, _EMB), jnp.float32),
            pltpu.VMEM((_G, _EMB), jnp.float32),
            pltpu.SemaphoreType.DMA,
            pltpu.SemaphoreType.DMA,
        ],
    )(idx, table)


_BLK = 256


def _mlp_body(pool_ref, sem_ref, wph_ref, bph_ref, wse_ref, bse_ref,
              wc1_ref, wc2_ref, bc_ref, wo_ref, bo_ref, out_ref):
    pool = pool_ref[...] * _INV_L
    dn = (((1,), (1,)), ((), ()))
    ph = lax.dot_general(pool, wph_ref[...], dn,
                         preferred_element_type=jnp.float32) + bph_ref[...]
    se = lax.dot_general(sem_ref[...], wse_ref[...], dn,
                         preferred_element_type=jnp.float32) + bse_ref[...]
    x = (lax.dot_general(ph, wc1_ref[...], dn,
                         preferred_element_type=jnp.float32)
         + lax.dot_general(se, wc2_ref[...], dn,
                           preferred_element_type=jnp.float32)
         + bc_ref[...])
    x = jnp.maximum(x, 0.0)
    o = jnp.sum(x * wo_ref[...], axis=1, keepdims=True) + bo_ref[...]
    out_ref[...] = jax.nn.sigmoid(o)


@jax.jit
def _mlp(pooled, semantic, W_ph, b_ph, W_se, b_se, wc1, wc2, b_c, W_o, b_o):
    n_blk = _B // _BLK
    full = lambda shape: pl.BlockSpec(shape, lambda i: (0, 0))
    return pl.pallas_call(
        _mlp_body,
        grid=(n_blk,),
        in_specs=[
            pl.BlockSpec((_BLK, _EMB), lambda i: (i, 0)),
            pl.BlockSpec((_BLK, 256), lambda i: (i, 0)),
            full((128, _EMB)),
            full((1, 128)),
            full((128, 256)),
            full((1, 128)),
            full((64, 128)),
            full((64, 128)),
            full((1, 64)),
            full((1, 64)),
            full((1, 1)),
        ],
        out_specs=pl.BlockSpec((_BLK, 1), lambda i: (i, 0)),
        out_shape=jax.ShapeDtypeStruct((_B, 1), jnp.float32),
    )(pooled, semantic, W_ph, b_ph, W_se, b_se, wc1, wc2, b_c, W_o, b_o)


def kernel(phonetic_token, semantic_embed, emb_table,
           W_ph, b_ph, W_se, b_se, W_c, b_c, W_o, b_o):
    idx = phonetic_token.astype(jnp.int32).reshape(_B * _CPR, _CHUNK)
    pooled = _pool(idx, emb_table)
    return _mlp(pooled, semantic_embed,
                W_ph, b_ph.reshape(1, -1),
                W_se, b_se.reshape(1, -1),
                W_c[:, :128], W_c[:, 128:], b_c.reshape(1, -1),
                W_o, b_o.reshape(1, -1))
